# baseline (speedup 1.0000x reference)
import functools

import jax
import jax.numpy as jnp
from jax import lax
from jax.experimental import pallas as pl
from jax.experimental.pallas import tpu as pltpu
from jax.experimental.pallas import tpu_sc as plsc


def _emb_call(B, V, D, NC, NS):
    NW = NC * NS
    b_per_w = B // NW
    V2 = V // 2
    CH = 128
    n_ch = b_per_w // CH
    mesh = plsc.VectorSubcoreMesh(core_axis_name="c", subcore_axis_name="s")

    @functools.partial(
        pl.kernel,
        mesh=mesh,
        out_type=jax.ShapeDtypeStruct((B, D), jnp.float32),
        scratch_types=[
            pltpu.VMEM((b_per_w,), jnp.int32),
            pltpu.VMEM((CH,), jnp.int32),
            pltpu.VMEM((CH,), jnp.int32),
            pltpu.VMEM((CH, D), jnp.float32),
            pltpu.VMEM((CH, D), jnp.float32),
            pltpu.VMEM((CH, D), jnp.float32),
            pltpu.SemaphoreType.DMA,
        ],
        compiler_params=pltpu.CompilerParams(use_tc_tiling_on_sc=False),
    )
    def emb(idx_hbm, tbl_a, tbl_b, out_hbm, idx_v, ia_v, ib_v, ra_v, rb_v, om_v, sem):
        wid = lax.axis_index("s") * NC + lax.axis_index("c")
        base = wid * b_per_w
        pltpu.sync_copy(idx_hbm.at[wid], idx_v)

        def chunk(c, carry):
            for g in range(CH // 16):
                v = idx_v[pl.ds(c * CH + g * 16, 16)]
                in_b = v >= V2
                ia_v[pl.ds(g * 16, 16)] = jnp.where(in_b, 0, v)
                ib_v[pl.ds(g * 16, 16)] = jnp.where(in_b, v - V2, 0)
            ca = pltpu.async_copy(tbl_a.at[ia_v], ra_v, sem)
            cb = pltpu.async_copy(tbl_b.at[ib_v], rb_v, sem)
            ca.wait()
            cb.wait()
            for g in range(CH // 16):
                v = idx_v[pl.ds(c * CH + g * 16, 16)]
                for b in range(16):
                    r = g * 16 + b
                    m = v[b] >= V2
                    for q in range(D // 16):
                        s = pl.ds(q * 16, 16)
                        om_v[r, s] = jnp.where(m, rb_v[r, s], ra_v[r, s])
            pltpu.sync_copy(om_v, out_hbm.at[pl.ds(base + c * CH, CH)])
            return carry

        lax.fori_loop(0, n_ch, chunk, 0)

    return emb


def kernel(batch, embedding_table):
    (B,) = batch.shape
    V, D = embedding_table.shape
    info = plsc.get_sparse_core_info()
    NC, NS = info.num_cores, info.num_subcores
    NW = NC * NS
    idx = batch.astype(jnp.int32).reshape(NW, B // NW)
    tbl_a = embedding_table[: V // 2]
    tbl_b = embedding_table[V // 2 :]
    return _emb_call(B, V, D, NC, NS)(idx, tbl_a, tbl_b)


# speedup vs baseline: 2.2380x; 2.2380x over previous
import functools

import jax
import jax.numpy as jnp
from jax import lax
from jax.experimental import pallas as pl
from jax.experimental.pallas import tpu as pltpu
from jax.experimental.pallas import tpu_sc as plsc


def _emb_call(B, D, NC, NS):
    NW = NC * NS
    b_per_w = B // NW
    H = b_per_w // 2
    mesh = plsc.VectorSubcoreMesh(core_axis_name="c", subcore_axis_name="s")

    @functools.partial(
        pl.kernel,
        mesh=mesh,
        out_type=jax.ShapeDtypeStruct((B, D), jnp.float32),
        scratch_types=[
            pltpu.VMEM((b_per_w,), jnp.int32),
            pltpu.VMEM((H, D), jnp.float32),
            pltpu.SemaphoreType.DMA,
            pltpu.SemaphoreType.DMA,
        ],
    )
    def emb(idx_hbm, table_hbm, out_hbm, idx_v, rows_v, sem_v, sem_h):
        wid = lax.axis_index("s") * NC + lax.axis_index("c")
        base = wid * b_per_w
        pltpu.sync_copy(idx_hbm.at[wid], idx_v)

        def body(g, carry):
            vec = idx_v[pl.ds(g * 16, 16)]
            for b in range(16):
                i = g * 16 + b
                pltpu.make_async_copy(
                    table_hbm.at[pl.ds(vec[b], 1)],
                    out_hbm.at[pl.ds(base + i, 1)],
                    sem_h,
                ).start()
            return carry

        def body2(g, carry):
            vec = idx_v[pl.ds(H + g * 16, 16)]
            for b in range(16):
                i = g * 16 + b
                pltpu.make_async_copy(
                    table_hbm.at[pl.ds(vec[b], 1)],
                    rows_v.at[pl.ds(i, 1)],
                    sem_v,
                ).start()
            return carry

        def both(g, carry):
            body(g, carry)
            body2(g, carry)
            return carry

        lax.fori_loop(0, H // 16, both, 0)
        pltpu.make_async_copy(table_hbm.at[pl.ds(0, H)], rows_v, sem_v).wait()
        pltpu.sync_copy(rows_v, out_hbm.at[pl.ds(base + H, H)])
        pltpu.make_async_copy(
            table_hbm.at[pl.ds(0, H)],
            out_hbm.at[pl.ds(base, H)],
            sem_h,
        ).wait()

    return emb


def kernel(batch, embedding_table):
    (B,) = batch.shape
    _, D = embedding_table.shape
    info = plsc.get_sparse_core_info()
    NC, NS = info.num_cores, info.num_subcores
    NW = NC * NS
    idx = batch.astype(jnp.int32).reshape(NW, B // NW)
    return _emb_call(B, D, NC, NS)(idx, embedding_table)
